# baseline (speedup 1.0000x reference)
import functools

import jax
import jax.numpy as jnp
from jax import lax
from jax.experimental import pallas as pl
from jax.experimental.pallas import tpu as pltpu
from jax.experimental.pallas import tpu_sc as plsc

N = 10000
E = 320000
D = 128
H = 16
C = 16

NC = 2
NS = 16
LANES = 16

NW = NC * NS
EPW = E // NW
BATCH = 1000
NBATCH = EPW // BATCH
NPAD = 10240
RPT = NPAD // NS

NP8 = N // 8
NPAD8 = NPAD * H // 128

F32 = jnp.float32


NBUF = 6


def _seg_body(with_deg, y_hbm, src_hbm, dst_hbm, z16_hbm, z1_hbm, ones_hbm, *rest):
    if with_deg:
        (part_out, deg_out, src_v, dst_v, ones_v, acc, dacc,
         rows, gsem, ssem, dsem) = rest
    else:
        part_out, src_v, dst_v, ones_v, acc, rows, gsem, ssem, dsem = rest
    cid = lax.axis_index("c")
    sid = lax.axis_index("s")
    w = sid * NC + cid

    cp_src = pltpu.async_copy(src_hbm.at[w], src_v, gsem[0])
    cp_dst = pltpu.async_copy(dst_hbm.at[w], dst_v, gsem[1])
    cp_z = pltpu.async_copy(
        z16_hbm, acc.at[pl.ds(sid * RPT, RPT)], ssem[0])
    if with_deg:
        cp_z1 = pltpu.async_copy(z1_hbm, dacc.at[pl.ds(sid * RPT, RPT)], ssem[1])
        cp_o = pltpu.async_copy(ones_hbm.at[pl.ds(0, BATCH)], ones_v, ssem[2])

    gd = [None] * NBATCH
    sd = [None] * NBATCH
    dd = [None] * NBATCH

    cp_src.wait()
    for j in range(min(NBUF, NBATCH)):
        gd[j] = pltpu.async_copy(y_hbm.at[src_v.at[j]], rows[j], gsem[j])
    cp_dst.wait()
    cp_z.wait()
    if with_deg:
        cp_z1.wait()
        cp_o.wait()
    plsc.subcore_barrier()

    for j in range(NBATCH):
        if j >= 1 and (j - 1) + NBUF < NBATCH:
            sd[j - 1].wait()
            k = (j - 1) % NBUF
            gd[j - 1 + NBUF] = pltpu.async_copy(
                y_hbm.at[src_v.at[j - 1 + NBUF]], rows[k], gsem[k])
        gd[j].wait()
        sd[j] = pltpu.async_copy(rows[j % NBUF], acc.at[dst_v.at[j]],
                                 ssem[j % NBUF], add=True)
        if with_deg:
            dd[j] = pltpu.async_copy(ones_v, dacc.at[dst_v.at[j]], dsem, add=True)
    for j in range(NBATCH):
        if sd[j] is not None and j + NBUF >= NBATCH:
            sd[j].wait()
    if with_deg:
        for j in range(NBATCH):
            dd[j].wait()
    plsc.subcore_barrier()

    sl = pl.ds(sid * RPT, RPT)
    pltpu.sync_copy(acc.at[sl], part_out.at[cid, sl])
    if with_deg:
        pltpu.sync_copy(dacc.at[sl], deg_out.at[cid, sl])


def _make_seg(with_deg):
    out_type = [jax.ShapeDtypeStruct((NC, NPAD, H), F32)]
    scratch = [
        pltpu.VMEM((NBATCH, BATCH), jnp.int32),
        pltpu.VMEM((NBATCH, BATCH), jnp.int32),
        pltpu.VMEM((BATCH,), F32),
    ]
    if with_deg:
        out_type.append(jax.ShapeDtypeStruct((NC, NPAD), F32))
    scratch.append(pltpu.VMEM_SHARED((NPAD, H), F32))
    if with_deg:
        scratch.append(pltpu.VMEM_SHARED((NPAD,), F32))
    scratch.append([pltpu.VMEM((BATCH, H), F32) for _ in range(NBUF)])
    scratch.append([pltpu.SemaphoreType.DMA for _ in range(NBUF)])
    scratch.append([pltpu.SemaphoreType.DMA for _ in range(NBUF)])
    scratch.append(pltpu.SemaphoreType.DMA)
    return pl.kernel(
        functools.partial(_seg_body, with_deg),
        out_type=out_type,
        mesh=plsc.VectorSubcoreMesh(
            core_axis_name="c", subcore_axis_name="s",
            num_cores=NC, num_subcores=NS,
        ),
        scratch_types=scratch,
        compiler_params=pltpu.CompilerParams(use_tc_tiling_on_sc=False),
    )


def _selector():
    lane_blk = lax.broadcasted_iota(jnp.int32, (8, 128), 1) // H
    sub = lax.broadcasted_iota(jnp.int32, (8, 128), 0)
    return (lane_blk == sub).astype(F32)


def _tile8_cols(w):
    return jnp.concatenate([w] * 8, axis=1)


def _blockdiag(w):
    big = jnp.concatenate([_tile8_cols(w)] * 8, axis=0)
    row_blk = lax.broadcasted_iota(jnp.int32, (128, 128), 0) // H
    col_blk = lax.broadcasted_iota(jnp.int32, (128, 128), 1) // H
    return jnp.where(row_blk == col_blk, big, 0.0)


def _mm1_body(x_ref, e_ref, wlT_ref, wrT_ref, b_ref, y_ref, r_ref, s_ref, d_ref,
              z16_ref, z1_ref, o1_ref):
    xv = x_ref[...]
    m = _selector()[None]
    dn = (((1,), (1,)), ((), ()))
    wlt = jnp.concatenate([wlT_ref[...]] * 8, axis=0)
    wrt = jnp.concatenate([wrT_ref[...]] * 8, axis=0)
    z = lax.dot_general(xv, wlt, dn, preferred_element_type=F32)
    y_ref[...] = jnp.sum(z.reshape(NP8, 8, 128) * m, axis=1)
    zr = lax.dot_general(xv, wrt, dn, preferred_element_type=F32)
    r_ref[...] = jnp.sum(zr.reshape(NP8, 8, 128) * m, axis=1) + _tile8_cols(b_ref[...])
    ev = e_ref[...]
    s_ref[...] = ev[0:1, :].reshape(E // 128, 128)
    d_ref[...] = ev[1:2, :].reshape(E // 128, 128)
    z16_ref[...] = jnp.zeros((RPT * H // 128, 128), F32)
    z1_ref[...] = jnp.zeros((RPT // 128, 128), F32)
    o1_ref[...] = jnp.ones((16, 128), F32)


def _mid_body(part_ref, deg_ref, r1_ref, wl_ref, wr_ref, b_ref, y_ref, r_ref):
    d8 = deg_ref[0, :NP8, :] + deg_ref[1, :NP8, :]
    d = jnp.maximum(jnp.dot(d8, _selector(), preferred_element_type=F32), 1.0)
    p = part_ref[0, :NP8, :] + part_ref[1, :NP8, :]
    h = jnp.maximum(p / d + r1_ref[...], 0.0)
    y_ref[...] = jnp.dot(h, _blockdiag(wl_ref[...]), preferred_element_type=F32)
    r_ref[...] = (jnp.dot(h, _blockdiag(wr_ref[...]), preferred_element_type=F32)
                  + _tile8_cols(b_ref[...]))


def _fin_body(part_ref, deg_ref, r2_ref, out_ref):
    d8 = deg_ref[0, :NP8, :] + deg_ref[1, :NP8, :]
    d = jnp.maximum(jnp.dot(d8, _selector(), preferred_element_type=F32), 1.0)
    p = part_ref[0, :NP8, :] + part_ref[1, :NP8, :]
    out_ref[...] = p / d + r2_ref[...]


_mm1 = pl.pallas_call(
    _mm1_body,
    out_shape=[
        jax.ShapeDtypeStruct((NP8, 128), F32),
        jax.ShapeDtypeStruct((NP8, 128), F32),
        jax.ShapeDtypeStruct((E // 128, 128), jnp.int32),
        jax.ShapeDtypeStruct((E // 128, 128), jnp.int32),
        jax.ShapeDtypeStruct((RPT * H // 128, 128), F32),
        jax.ShapeDtypeStruct((RPT // 128, 128), F32),
        jax.ShapeDtypeStruct((16, 128), F32),
    ],
)

_mid = pl.pallas_call(
    _mid_body,
    out_shape=[jax.ShapeDtypeStruct((NP8, 128), F32), jax.ShapeDtypeStruct((NP8, 128), F32)],
)

_fin = pl.pallas_call(
    _fin_body,
    out_shape=jax.ShapeDtypeStruct((NP8, 128), F32),
)


def kernel(x, edge_index, W1_l, b1, W1_r, W2_l, b2, W2_r):
    y1p, r1p, srcp, dstp, z16p, z1p, o1p = _mm1(
        x, edge_index, W1_l.T, W1_r.T, b1.reshape(1, H))
    src = srcp.reshape(NW, NBATCH, BATCH)
    dst = dstp.reshape(NW, NBATCH, BATCH)
    z16 = z16p.reshape(RPT, H)
    z1 = z1p.reshape(RPT)
    o1 = o1p.reshape(2048)

    seg1 = _make_seg(True)
    part1, degp = seg1(y1p.reshape(N, H), src, dst, z16, z1, o1)
    deg8 = degp.reshape(NC, NPAD8, 8)

    y2p, r2p = _mid(part1.reshape(NC, NPAD8, 128), deg8,
                    r1p, W2_l, W2_r, b2.reshape(1, C))

    seg2 = _make_seg(False)
    (part2,) = seg2(y2p.reshape(N, H), src, dst, z16, z1, o1)

    outp = _fin(part2.reshape(NC, NPAD8, 128), deg8, r2p)
    return outp.reshape(N, C)


# speedup vs baseline: 1.0374x; 1.0374x over previous
import functools

import jax
import jax.numpy as jnp
from jax import lax
from jax.experimental import pallas as pl
from jax.experimental.pallas import tpu as pltpu
from jax.experimental.pallas import tpu_sc as plsc

N = 10000
E = 320000
D = 128
H = 16
C = 16

NC = 2
NS = 16
LANES = 16

NW = NC * NS
EPW = E // NW
BATCH = 400
NBATCH = EPW // BATCH
NPAD = 10240
RPT = NPAD // NS

NP8 = N // 8
NPAD8 = NPAD * H // 128

F32 = jnp.float32


NBUF = 5


def _seg_body(with_deg, y_hbm, src_hbm, dst_hbm, z16_hbm, z1_hbm, ones_hbm, *rest):
    if with_deg:
        (part_out, deg_out, src_v, dst_v, ones_v, acc, dacc,
         rows, gsem, ssem, dsem) = rest
    else:
        part_out, src_v, dst_v, ones_v, acc, rows, gsem, ssem, dsem = rest
    cid = lax.axis_index("c")
    sid = lax.axis_index("s")
    w = sid * NC + cid

    cp_src = pltpu.async_copy(src_hbm.at[w], src_v, gsem[0])
    cp_dst = pltpu.async_copy(dst_hbm.at[w], dst_v, gsem[1])
    cp_z = pltpu.async_copy(
        z16_hbm, acc.at[pl.ds(sid * RPT, RPT)], ssem[0])
    if with_deg:
        cp_z1 = pltpu.async_copy(z1_hbm, dacc.at[pl.ds(sid * RPT, RPT)], ssem[1])
        cp_o = pltpu.async_copy(ones_hbm.at[pl.ds(0, BATCH)], ones_v, ssem[2])

    gd = [None] * NBATCH
    sd = [None] * NBATCH
    dd = [None] * NBATCH

    cp_src.wait()
    for j in range(min(NBUF, NBATCH)):
        gd[j] = pltpu.async_copy(y_hbm.at[src_v.at[j]], rows[j], gsem[j])
    cp_dst.wait()
    cp_z.wait()
    if with_deg:
        cp_z1.wait()
        cp_o.wait()
    plsc.subcore_barrier()

    for j in range(NBATCH):
        if j >= 1 and (j - 1) + NBUF < NBATCH:
            sd[j - 1].wait()
            k = (j - 1) % NBUF
            gd[j - 1 + NBUF] = pltpu.async_copy(
                y_hbm.at[src_v.at[j - 1 + NBUF]], rows[k], gsem[k])
        gd[j].wait()
        sd[j] = pltpu.async_copy(rows[j % NBUF], acc.at[dst_v.at[j]],
                                 ssem[j % NBUF], add=True)
        if with_deg:
            dd[j] = pltpu.async_copy(ones_v, dacc.at[dst_v.at[j]], dsem, add=True)
    for j in range(NBATCH):
        if sd[j] is not None and j + NBUF >= NBATCH:
            sd[j].wait()
    if with_deg:
        for j in range(NBATCH):
            dd[j].wait()
    plsc.subcore_barrier()

    sl = pl.ds(sid * RPT, RPT)
    pltpu.sync_copy(acc.at[sl], part_out.at[cid, sl])
    if with_deg:
        pltpu.sync_copy(dacc.at[sl], deg_out.at[cid, sl])


def _make_seg(with_deg):
    out_type = [jax.ShapeDtypeStruct((NC, NPAD, H), F32)]
    scratch = [
        pltpu.VMEM((NBATCH, BATCH), jnp.int32),
        pltpu.VMEM((NBATCH, BATCH), jnp.int32),
        pltpu.VMEM((BATCH,), F32),
    ]
    if with_deg:
        out_type.append(jax.ShapeDtypeStruct((NC, NPAD), F32))
    scratch.append(pltpu.VMEM_SHARED((NPAD, H), F32))
    if with_deg:
        scratch.append(pltpu.VMEM_SHARED((NPAD,), F32))
    scratch.append([pltpu.VMEM((BATCH, H), F32) for _ in range(NBUF)])
    scratch.append([pltpu.SemaphoreType.DMA for _ in range(NBUF)])
    scratch.append([pltpu.SemaphoreType.DMA for _ in range(NBUF)])
    scratch.append(pltpu.SemaphoreType.DMA)
    return pl.kernel(
        functools.partial(_seg_body, with_deg),
        out_type=out_type,
        mesh=plsc.VectorSubcoreMesh(
            core_axis_name="c", subcore_axis_name="s",
            num_cores=NC, num_subcores=NS,
        ),
        scratch_types=scratch,
        compiler_params=pltpu.CompilerParams(use_tc_tiling_on_sc=False),
    )


def _selector():
    lane_blk = lax.broadcasted_iota(jnp.int32, (8, 128), 1) // H
    sub = lax.broadcasted_iota(jnp.int32, (8, 128), 0)
    return (lane_blk == sub).astype(F32)


def _tile8_cols(w):
    return jnp.concatenate([w] * 8, axis=1)


def _blockdiag(w):
    big = jnp.concatenate([_tile8_cols(w)] * 8, axis=0)
    row_blk = lax.broadcasted_iota(jnp.int32, (128, 128), 0) // H
    col_blk = lax.broadcasted_iota(jnp.int32, (128, 128), 1) // H
    return jnp.where(row_blk == col_blk, big, 0.0)


def _mm1_body(x_ref, e_ref, wlT_ref, wrT_ref, b_ref, y_ref, r_ref, s_ref, d_ref,
              z16_ref, z1_ref, o1_ref):
    xv = x_ref[...]
    m = _selector()[None]
    dn = (((1,), (1,)), ((), ()))
    wlt = jnp.concatenate([wlT_ref[...]] * 8, axis=0)
    wrt = jnp.concatenate([wrT_ref[...]] * 8, axis=0)
    z = lax.dot_general(xv, wlt, dn, preferred_element_type=F32)
    y_ref[...] = jnp.sum(z.reshape(NP8, 8, 128) * m, axis=1)
    zr = lax.dot_general(xv, wrt, dn, preferred_element_type=F32)
    r_ref[...] = jnp.sum(zr.reshape(NP8, 8, 128) * m, axis=1) + _tile8_cols(b_ref[...])
    ev = e_ref[...]
    s_ref[...] = ev[0:1, :].reshape(E // 128, 128)
    d_ref[...] = ev[1:2, :].reshape(E // 128, 128)
    z16_ref[...] = jnp.zeros((RPT * H // 128, 128), F32)
    z1_ref[...] = jnp.zeros((RPT // 128, 128), F32)
    o1_ref[...] = jnp.ones((16, 128), F32)


def _mid_body(part_ref, deg_ref, r1_ref, wl_ref, wr_ref, b_ref, y_ref, r_ref):
    d8 = deg_ref[0, :NP8, :] + deg_ref[1, :NP8, :]
    d = jnp.maximum(jnp.dot(d8, _selector(), preferred_element_type=F32), 1.0)
    p = part_ref[0, :NP8, :] + part_ref[1, :NP8, :]
    h = jnp.maximum(p / d + r1_ref[...], 0.0)
    y_ref[...] = jnp.dot(h, _blockdiag(wl_ref[...]), preferred_element_type=F32)
    r_ref[...] = (jnp.dot(h, _blockdiag(wr_ref[...]), preferred_element_type=F32)
                  + _tile8_cols(b_ref[...]))


def _fin_body(part_ref, deg_ref, r2_ref, out_ref):
    d8 = deg_ref[0, :NP8, :] + deg_ref[1, :NP8, :]
    d = jnp.maximum(jnp.dot(d8, _selector(), preferred_element_type=F32), 1.0)
    p = part_ref[0, :NP8, :] + part_ref[1, :NP8, :]
    out_ref[...] = p / d + r2_ref[...]


_mm1 = pl.pallas_call(
    _mm1_body,
    out_shape=[
        jax.ShapeDtypeStruct((NP8, 128), F32),
        jax.ShapeDtypeStruct((NP8, 128), F32),
        jax.ShapeDtypeStruct((E // 128, 128), jnp.int32),
        jax.ShapeDtypeStruct((E // 128, 128), jnp.int32),
        jax.ShapeDtypeStruct((RPT * H // 128, 128), F32),
        jax.ShapeDtypeStruct((RPT // 128, 128), F32),
        jax.ShapeDtypeStruct((16, 128), F32),
    ],
)

_mid = pl.pallas_call(
    _mid_body,
    out_shape=[jax.ShapeDtypeStruct((NP8, 128), F32), jax.ShapeDtypeStruct((NP8, 128), F32)],
)

_fin = pl.pallas_call(
    _fin_body,
    out_shape=jax.ShapeDtypeStruct((NP8, 128), F32),
)


def kernel(x, edge_index, W1_l, b1, W1_r, W2_l, b2, W2_r):
    y1p, r1p, srcp, dstp, z16p, z1p, o1p = _mm1(
        x, edge_index, W1_l.T, W1_r.T, b1.reshape(1, H))
    src = srcp.reshape(NW, NBATCH, BATCH)
    dst = dstp.reshape(NW, NBATCH, BATCH)
    z16 = z16p.reshape(RPT, H)
    z1 = z1p.reshape(RPT)
    o1 = o1p.reshape(2048)

    seg1 = _make_seg(True)
    part1, degp = seg1(y1p.reshape(N, H), src, dst, z16, z1, o1)
    deg8 = degp.reshape(NC, NPAD8, 8)

    y2p, r2p = _mid(part1.reshape(NC, NPAD8, 128), deg8,
                    r1p, W2_l, W2_r, b2.reshape(1, C))

    seg2 = _make_seg(False)
    (part2,) = seg2(y2p.reshape(N, H), src, dst, z16, z1, o1)

    outp = _fin(part2.reshape(NC, NPAD8, 128), deg8, r2p)
    return outp.reshape(N, C)
